# Initial kernel scaffold; baseline (speedup 1.0000x reference)
import jax
import jax.numpy as jnp
from jax.experimental import pallas as pl


def kernel(x, edge_index, W0, b0, W1, b1, Wg, bg):
    raise NotImplementedError("write your pallas kernel here")


# speedup vs baseline: 3.4289x; 3.4289x over previous
import functools

import jax
import jax.numpy as jnp
from jax import lax
from jax.experimental import pallas as pl
from jax.experimental.pallas import tpu as pltpu
from jax.experimental.pallas import tpu_sc as plsc

N = 10000
E = 160000
D_IN = 256
H = 512
D_OUT = 256

NC = 2
NS = 16
L = 16

EPT = E // NS
BATCH = 128
GRP = 8
NGRP = 10
NB = GRP * NGRP
EPT_PAD = NB * BATCH
R_ACC = 10112
RPT = R_ACC // NS

EPW = E // (NC * NS)
NV = EPW // L

_mesh = plsc.VectorSubcoreMesh(core_axis_name="c", subcore_axis_name="s")


def _sc_degrees(sidx_both, ones_rows, z16):
    @functools.partial(
        pl.kernel,
        out_type=jax.ShapeDtypeStruct((2, R_ACC, 128), jnp.float32),
        mesh=_mesh,
        scratch_types=[
            pltpu.VMEM((GRP, BATCH), jnp.int32),
            pltpu.VMEM((BATCH, 128), jnp.float32),
            pltpu.VMEM_SHARED((R_ACC, 128), jnp.float32),
        ],
    )
    def deg_kernel(sidx_hbm, ones_hbm, z_hbm, degp, sbuf, ones_v, acc):
        c = lax.axis_index("c")
        s = lax.axis_index("s")
        pltpu.sync_copy(z_hbm, acc.at[pl.ds(s * RPT, RPT)])
        pltpu.sync_copy(ones_hbm, ones_v)
        plsc.subcore_barrier()

        def body(g, carry):
            pltpu.sync_copy(sidx_hbm.at[c, s, pl.ds(g * GRP, GRP)], sbuf)
            for j in range(GRP):
                pltpu.sync_copy(ones_v, acc.at[sbuf.at[j]], add=True)
            return carry

        lax.fori_loop(0, NGRP, body, 0)
        plsc.subcore_barrier()
        pltpu.sync_copy(acc.at[pl.ds(s * RPT, RPT)],
                        degp.at[c, pl.ds(s * RPT, RPT)])

    return deg_kernel(sidx_both, ones_rows, z16)


def _make_agg_kernel(n_slabs, n_phases):
    @functools.partial(
        pl.kernel,
        out_type=jax.ShapeDtypeStruct((n_slabs, R_ACC, 128), jnp.float32),
        mesh=_mesh,
        scratch_types=[
            pltpu.VMEM((GRP, BATCH), jnp.int32),
            pltpu.VMEM((GRP, BATCH), jnp.int32),
            pltpu.VMEM((BATCH, 128), jnp.float32),
            pltpu.VMEM((BATCH, 128), jnp.float32),
            pltpu.VMEM_SHARED((R_ACC, 128), jnp.float32),
            pltpu.SemaphoreType.DMA,
            pltpu.SemaphoreType.DMA,
            pltpu.SemaphoreType.DMA,
            pltpu.SemaphoreType.DMA,
        ],
    )
    def agg_kernel(tab_hbm, gidx_hbm, sidx_hbm, z_hbm, out_hbm,
                   gidx, sidx, bufa, bufb, acc, gsa, gsb, ssa, ssb):
        c = lax.axis_index("c")
        s = lax.axis_index("s")
        for p in range(n_phases):
            pltpu.sync_copy(z_hbm, acc.at[pl.ds(s * RPT, RPT)])
            plsc.subcore_barrier()

            def group(g, carry):
                pltpu.sync_copy(gidx_hbm.at[c, p, s, pl.ds(g * GRP, GRP)], gidx)
                pltpu.sync_copy(sidx_hbm.at[s, pl.ds(g * GRP, GRP)], sidx)
                for j in range(GRP // 2):
                    b0 = j * 2
                    b1 = j * 2 + 1
                    ga = pltpu.async_copy(tab_hbm.at[gidx.at[b0]], bufa, gsa)
                    gb = pltpu.async_copy(tab_hbm.at[gidx.at[b1]], bufb, gsb)
                    ga.wait()
                    sa = pltpu.async_copy(bufa, acc.at[sidx.at[b0]], ssa,
                                          add=True)
                    gb.wait()
                    sb = pltpu.async_copy(bufb, acc.at[sidx.at[b1]], ssb,
                                          add=True)
                    sa.wait()
                    sb.wait()
                return carry

            lax.fori_loop(0, NGRP, group, 0)
            plsc.subcore_barrier()
            q = c * n_phases + p
            pltpu.sync_copy(acc.at[pl.ds(s * RPT, RPT)],
                            out_hbm.at[q, pl.ds(s * RPT, RPT)])
            if p + 1 < n_phases:
                plsc.subcore_barrier()

    return agg_kernel


_agg1_kernel = _make_agg_kernel(n_slabs=2, n_phases=1)
_agg2_kernel = _make_agg_kernel(n_slabs=4, n_phases=2)


BN = 400
NBLK = N // BN


def _tc_norms_xs(degp, x):
    def body(degp_ref, x_ref, xs_ref, norms_ref):
        dp = degp_ref[...]
        d = jnp.sum(dp, axis=2)
        nsrc = lax.rsqrt(jnp.maximum(d[0], 1.0))
        ndst = lax.rsqrt(jnp.maximum(d[1], 1.0))
        xs_ref[...] = x_ref[...] * nsrc[:, None]
        norms_ref[...] = jnp.stack([nsrc, ndst], axis=1)

    return pl.pallas_call(
        body,
        grid=(NBLK,),
        in_specs=[
            pl.BlockSpec((2, BN, 128), lambda i: (0, i, 0)),
            pl.BlockSpec((BN, D_IN), lambda i: (i, 0)),
        ],
        out_specs=[
            pl.BlockSpec((BN, D_IN), lambda i: (i, 0)),
            pl.BlockSpec((BN, 2), lambda i: (i, 0)),
        ],
        out_shape=[
            jax.ShapeDtypeStruct((N, D_IN), jnp.float32),
            jax.ShapeDtypeStruct((N, 2), jnp.float32),
        ],
    )(degp, x)


def _tc_layer1(agg1, norms, w0r, b0r):
    def body(agg_ref, norms_ref, w_ref, b_ref, out_ref):
        a = agg_ref[...]
        z = (jnp.dot(a[0], w_ref[0], preferred_element_type=jnp.float32)
             + jnp.dot(a[1], w_ref[1], preferred_element_type=jnp.float32))
        ns = norms_ref[:, 0]
        nd = norms_ref[:, 1]
        h = jnp.maximum(z * nd[:, None] + b_ref[...], 0.0)
        out_ref[...] = h * ns[:, None]

    return pl.pallas_call(
        body,
        grid=(NBLK,),
        in_specs=[
            pl.BlockSpec((2, BN, 128), lambda i: (0, i, 0)),
            pl.BlockSpec((BN, 2), lambda i: (i, 0)),
            pl.BlockSpec((2, 128, H), lambda i: (0, 0, 0)),
            pl.BlockSpec((1, H), lambda i: (0, 0)),
        ],
        out_specs=pl.BlockSpec((BN, H), lambda i: (i, 0)),
        out_shape=jax.ShapeDtypeStruct((N, H), jnp.float32),
    )(agg1, norms, w0r, b0r)


def _tc_layer2(agg2, norms, w1r, b1r, wg, bgr):
    def body(agg_ref, norms_ref, w_ref, b_ref, wg_ref, bg_ref, out_ref, acc):
        i = pl.program_id(0)
        a = agg_ref[...]
        z = jnp.dot(a[0], w_ref[0], preferred_element_type=jnp.float32)
        for q in range(1, 4):
            z += jnp.dot(a[q], w_ref[q], preferred_element_type=jnp.float32)
        nd = norms_ref[:, 1]
        h = jnp.maximum(z * nd[:, None] + b_ref[...], 0.0)
        part = jnp.sum(h, axis=0, keepdims=True)

        @pl.when(i == 0)
        def _():
            acc[...] = part

        @pl.when(i > 0)
        def _():
            acc[...] = acc[...] + part

        @pl.when(i == NBLK - 1)
        def _():
            out_ref[...] = (
                jnp.dot(acc[...] * (1.0 / N), wg_ref[...],
                        preferred_element_type=jnp.float32) + bg_ref[...])

    return pl.pallas_call(
        body,
        grid=(NBLK,),
        in_specs=[
            pl.BlockSpec((4, BN, 128), lambda i: (0, i, 0)),
            pl.BlockSpec((BN, 2), lambda i: (i, 0)),
            pl.BlockSpec((4, 128, H), lambda i: (0, 0, 0)),
            pl.BlockSpec((1, H), lambda i: (0, 0)),
            pl.BlockSpec((H, D_OUT), lambda i: (0, 0)),
            pl.BlockSpec((1, D_OUT), lambda i: (0, 0)),
        ],
        out_specs=pl.BlockSpec((1, D_OUT), lambda i: (0, 0)),
        out_shape=jax.ShapeDtypeStruct((1, D_OUT), jnp.float32),
        scratch_shapes=[pltpu.VMEM((1, H), jnp.float32)],
    )(agg2, norms, w1r, b1r, wg, bgr)


def kernel(x, edge_index, W0, b0, W1, b1, Wg, bg):
    src = edge_index[0]
    dst = edge_index[1]

    pad = EPT_PAD - EPT
    srcp = jnp.pad(src.reshape(NS, EPT), ((0, 0), (0, pad)))
    cols = jnp.arange(EPT_PAD, dtype=jnp.int32)
    valid = cols < EPT
    dstp = jnp.pad(dst.reshape(NS, EPT), ((0, 0), (0, pad)))
    dummy = N + lax.rem(cols, 16)[None, :]
    dstp = jnp.where(valid[None, :], dstp, dummy)
    sidx = dstp.reshape(NS, NB, BATCH)
    srcd = jnp.where(valid[None, :], srcp, dummy)
    sidx_both = jnp.stack([srcd, dstp]).reshape(NC, NS, NB, BATCH)

    g1 = jnp.stack([srcp * 2, srcp * 2 + 1])
    g1 = g1.reshape(NC, 1, NS, NB, BATCH)
    g2 = jnp.stack([srcp * 4, srcp * 4 + 1, srcp * 4 + 2, srcp * 4 + 3])
    g2 = g2.reshape(NC, 2, NS, NB, BATCH)

    ones_rows = jnp.zeros((BATCH, 128), jnp.float32).at[:, 0].set(1.0)
    zrp = jnp.zeros((RPT, 128), jnp.float32)

    degp = _sc_degrees(sidx_both, ones_rows, zrp)
    xs, norms = _tc_norms_xs(degp, x)
    agg1 = _agg1_kernel(xs.reshape(2 * N, 128), g1, sidx, zrp)
    h1s = _tc_layer1(agg1, norms, W0.reshape(2, 128, H), b0.reshape(1, H))
    agg2 = _agg2_kernel(h1s.reshape(4 * N, 128), g2, sidx, zrp)
    out = _tc_layer2(agg2, norms, W1.reshape(4, 128, H), b1.reshape(1, H),
                     Wg, bg.reshape(1, D_OUT))
    return out


# speedup vs baseline: 3.8141x; 1.1123x over previous
import functools

import jax
import jax.numpy as jnp
from jax import lax
from jax.experimental import pallas as pl
from jax.experimental.pallas import tpu as pltpu
from jax.experimental.pallas import tpu_sc as plsc

N = 10000
E = 160000
D_IN = 256
H = 512
D_OUT = 256

NC = 2
NS = 16
L = 16

EPT = E // NS
BATCH = 64
GRP = 16
NGRP = 10
NB = GRP * NGRP
EPT_PAD = NB * BATCH
NBUF = 4
R_ACC = 10112
RPT = R_ACC // NS

EPW = E // (NC * NS)
NV = EPW // L

_mesh = plsc.VectorSubcoreMesh(core_axis_name="c", subcore_axis_name="s")


RD = 10240
RPTD = RD // NS

def _sc_degrees(sidx_both, ones_vec, zd):
    @functools.partial(
        pl.kernel,
        out_type=jax.ShapeDtypeStruct((2, RD), jnp.float32),
        mesh=_mesh,
        scratch_types=[
            pltpu.VMEM((GRP, BATCH), jnp.int32),
            pltpu.VMEM((BATCH,), jnp.float32),
            pltpu.VMEM_SHARED((RD,), jnp.float32),
        ],
    )
    def deg_kernel(sidx_hbm, ones_hbm, z_hbm, degp, sbuf, ones_v, acc):
        c = lax.axis_index("c")
        s = lax.axis_index("s")
        pltpu.sync_copy(z_hbm, acc.at[pl.ds(s * RPTD, RPTD)])
        pltpu.sync_copy(ones_hbm, ones_v)
        plsc.subcore_barrier()

        def body(g, carry):
            pltpu.sync_copy(sidx_hbm.at[c, s, pl.ds(g * GRP, GRP)], sbuf)
            for j in range(GRP):
                pltpu.sync_copy(ones_v, acc.at[sbuf.at[j]], add=True)
            return carry

        lax.fori_loop(0, NGRP, body, 0)
        plsc.subcore_barrier()
        pltpu.sync_copy(acc.at[pl.ds(s * RPTD, RPTD)],
                        degp.at[c, pl.ds(s * RPTD, RPTD)])

    return deg_kernel(sidx_both, ones_vec, zd)


def _make_agg_kernel(n_slabs, n_phases):
    @functools.partial(
        pl.kernel,
        out_type=jax.ShapeDtypeStruct((n_slabs, R_ACC, 128), jnp.float32),
        mesh=_mesh,
        scratch_types=[
            pltpu.VMEM((GRP, BATCH), jnp.int32),
            pltpu.VMEM((GRP, BATCH), jnp.int32),
            [pltpu.VMEM((BATCH, 128), jnp.float32) for _ in range(NBUF)],
            [pltpu.SemaphoreType.DMA for _ in range(NBUF)],
            [pltpu.SemaphoreType.DMA for _ in range(NBUF)],
            pltpu.VMEM_SHARED((R_ACC, 128), jnp.float32),
        ],
    )
    def agg_kernel(tab_hbm, gidx_hbm, sidx_hbm, z_hbm, out_hbm,
                   gidx, sidx, bufs, gsems, ssems, acc):
        c = lax.axis_index("c")
        s = lax.axis_index("s")
        for p in range(n_phases):
            pltpu.sync_copy(z_hbm, acc.at[pl.ds(s * RPT, RPT)])
            plsc.subcore_barrier()

            def group(g, carry):
                pltpu.sync_copy(gidx_hbm.at[c, p, s, pl.ds(g * GRP, GRP)], gidx)
                pltpu.sync_copy(sidx_hbm.at[s, pl.ds(g * GRP, GRP)], sidx)
                gd = [None] * NBUF
                sd = [None] * NBUF
                for j in range(GRP):
                    k = j % NBUF
                    if sd[k] is not None:
                        sd[k].wait()
                    gd[k] = pltpu.async_copy(tab_hbm.at[gidx.at[j]], bufs[k],
                                             gsems[k])
                    if j >= 1:
                        kp = (j - 1) % NBUF
                        gd[kp].wait()
                        sd[kp] = pltpu.async_copy(bufs[kp],
                                                  acc.at[sidx.at[j - 1]],
                                                  ssems[kp], add=True)
                kl = (GRP - 1) % NBUF
                gd[kl].wait()
                sd[kl] = pltpu.async_copy(bufs[kl], acc.at[sidx.at[GRP - 1]],
                                          ssems[kl], add=True)
                for k in range(NBUF):
                    if sd[k] is not None:
                        sd[k].wait()
                return carry

            lax.fori_loop(0, NGRP, group, 0)
            plsc.subcore_barrier()
            q = c * n_phases + p
            pltpu.sync_copy(acc.at[pl.ds(s * RPT, RPT)],
                            out_hbm.at[q, pl.ds(s * RPT, RPT)])
            if p + 1 < n_phases:
                plsc.subcore_barrier()

    return agg_kernel


_agg1_kernel = _make_agg_kernel(n_slabs=2, n_phases=1)
_agg2_kernel = _make_agg_kernel(n_slabs=4, n_phases=2)


BN = 512
NBLK = (N + BN - 1) // BN
RB = BN // 8


def _tc_norms_xs(degp, x):
    def body(degp_ref, x_ref, xs_ref, ns_ref, nd_ref):
        dp = degp_ref[...]
        nsrc = lax.rsqrt(jnp.maximum(dp[0], 1.0))
        ndst = lax.rsqrt(jnp.maximum(dp[1], 1.0))
        x3 = x_ref[...].reshape(RB, 8, D_IN)
        xs_ref[...] = (x3 * nsrc[:, :, None]).reshape(BN, D_IN)
        ns_ref[...] = nsrc
        nd_ref[...] = ndst

    return pl.pallas_call(
        body,
        grid=(NBLK,),
        in_specs=[
            pl.BlockSpec((2, RB, 8), lambda i: (0, i, 0)),
            pl.BlockSpec((BN, D_IN), lambda i: (i, 0)),
        ],
        out_specs=[
            pl.BlockSpec((BN, D_IN), lambda i: (i, 0)),
            pl.BlockSpec((RB, 8), lambda i: (i, 0)),
            pl.BlockSpec((RB, 8), lambda i: (i, 0)),
        ],
        out_shape=[
            jax.ShapeDtypeStruct((N, D_IN), jnp.float32),
            jax.ShapeDtypeStruct((RD // 8, 8), jnp.float32),
            jax.ShapeDtypeStruct((RD // 8, 8), jnp.float32),
        ],
    )(degp, x)


def _tc_layer1(agg1, ns, nd, w0r, b0r):
    def body(agg_ref, ns_ref, nd_ref, w_ref, b_ref, out_ref):
        a = agg_ref[...]
        z = (jnp.dot(a[0], w_ref[0], preferred_element_type=jnp.float32)
             + jnp.dot(a[1], w_ref[1], preferred_element_type=jnp.float32))
        z3 = z.reshape(RB, 8, H)
        h3 = jnp.maximum(z3 * nd_ref[...][:, :, None]
                         + b_ref[...].reshape(1, 1, H), 0.0)
        out_ref[...] = (h3 * ns_ref[...][:, :, None]).reshape(BN, H)

    return pl.pallas_call(
        body,
        grid=(NBLK,),
        in_specs=[
            pl.BlockSpec((2, BN, 128), lambda i: (0, i, 0)),
            pl.BlockSpec((RB, 8), lambda i: (i, 0)),
            pl.BlockSpec((RB, 8), lambda i: (i, 0)),
            pl.BlockSpec((2, 128, H), lambda i: (0, 0, 0)),
            pl.BlockSpec((1, H), lambda i: (0, 0)),
        ],
        out_specs=pl.BlockSpec((BN, H), lambda i: (i, 0)),
        out_shape=jax.ShapeDtypeStruct((N, H), jnp.float32),
    )(agg1, ns, nd, w0r, b0r)


def _tc_layer2(agg2, nd, w1r, b1r, wg, bgr):
    def body(agg_ref, nd_ref, w_ref, b_ref, wg_ref, bg_ref, out_ref, acc):
        i = pl.program_id(0)
        a = agg_ref[...]
        z = jnp.dot(a[0], w_ref[0], preferred_element_type=jnp.float32)
        for q in range(1, 4):
            z += jnp.dot(a[q], w_ref[q], preferred_element_type=jnp.float32)
        z3 = z.reshape(RB, 8, H)
        h3 = jnp.maximum(z3 * nd_ref[...][:, :, None]
                         + b_ref[...].reshape(1, 1, H), 0.0)
        h = h3.reshape(BN, H)
        row = lax.broadcasted_iota(jnp.int32, (BN, 1), 0)
        h = jnp.where(row < N - i * BN, h, 0.0)
        part = jnp.sum(h, axis=0, keepdims=True)

        @pl.when(i == 0)
        def _():
            acc[...] = part

        @pl.when(i > 0)
        def _():
            acc[...] = acc[...] + part

        @pl.when(i == NBLK - 1)
        def _():
            out_ref[...] = (
                jnp.dot(acc[...] * (1.0 / N), wg_ref[...],
                        preferred_element_type=jnp.float32) + bg_ref[...])

    return pl.pallas_call(
        body,
        grid=(NBLK,),
        in_specs=[
            pl.BlockSpec((4, BN, 128), lambda i: (0, i, 0)),
            pl.BlockSpec((RB, 8), lambda i: (i, 0)),
            pl.BlockSpec((4, 128, H), lambda i: (0, 0, 0)),
            pl.BlockSpec((1, H), lambda i: (0, 0)),
            pl.BlockSpec((H, D_OUT), lambda i: (0, 0)),
            pl.BlockSpec((1, D_OUT), lambda i: (0, 0)),
        ],
        out_specs=pl.BlockSpec((1, D_OUT), lambda i: (0, 0)),
        out_shape=jax.ShapeDtypeStruct((1, D_OUT), jnp.float32),
        scratch_shapes=[pltpu.VMEM((1, H), jnp.float32)],
    )(agg2, nd, w1r, b1r, wg, bgr)


def kernel(x, edge_index, W0, b0, W1, b1, Wg, bg):
    src = edge_index[0]
    dst = edge_index[1]

    pad = EPT_PAD - EPT
    srcp = jnp.pad(src.reshape(NS, EPT), ((0, 0), (0, pad)))
    cols = jnp.arange(EPT_PAD, dtype=jnp.int32)
    valid = cols < EPT
    dstp = jnp.pad(dst.reshape(NS, EPT), ((0, 0), (0, pad)))
    dummy = N + lax.rem(cols, 16)[None, :]
    dstp = jnp.where(valid[None, :], dstp, dummy)
    sidx = dstp.reshape(NS, NB, BATCH)
    srcd = jnp.where(valid[None, :], srcp, dummy)
    sidx_both = jnp.stack([srcd, dstp]).reshape(NC, NS, NB, BATCH)

    g1 = jnp.stack([srcp * 2, srcp * 2 + 1])
    g1 = g1.reshape(NC, 1, NS, NB, BATCH)
    g2 = jnp.stack([srcp * 4, srcp * 4 + 1, srcp * 4 + 2, srcp * 4 + 3])
    g2 = g2.reshape(NC, 2, NS, NB, BATCH)

    ones_vec = jnp.ones((BATCH,), jnp.float32)
    zd = jnp.zeros((RPTD,), jnp.float32)
    zrp = jnp.zeros((RPT, 128), jnp.float32)

    degp = _sc_degrees(sidx_both, ones_vec, zd)
    xs, ns, nd = _tc_norms_xs(degp.reshape(2, RD // 8, 8), x)
    agg1 = _agg1_kernel(xs.reshape(2 * N, 128), g1, sidx, zrp)
    h1s = _tc_layer1(agg1, ns, nd, W0.reshape(2, 128, H), b0.reshape(1, H))
    agg2 = _agg2_kernel(h1s.reshape(4 * N, 128), g2, sidx, zrp)
    out = _tc_layer2(agg2, nd, W1.reshape(4, 128, H), b1.reshape(1, H),
                     Wg, bg.reshape(1, D_OUT))
    return out
